# baseline (speedup 1.0000x reference)
import jax
import jax.numpy as jnp
from jax.experimental import pallas as pl
from jax.experimental.pallas import tpu as pltpu

_ROWS = 128
_N = 32768
_BLK = 16384
_SUB = 128


def _body(x_ref, t_ref, o_ref, carry_ref):
    i = pl.program_id(0)

    @pl.when(i == 0)
    def _init():
        carry_ref[...] = jnp.zeros_like(carry_ref)

    carry = carry_ref[...]
    for k in range(_BLK // _SUB):
        x = x_ref[:, k * _SUB:(k + 1) * _SUB]
        pre = jax.lax.dot(
            x.astype(jnp.bfloat16), t_ref[...],
            preferred_element_type=jnp.float32)
        o_ref[:, k * _SUB:(k + 1) * _SUB] = pre + carry
        carry = carry + jnp.sum(x, axis=1, keepdims=True)
    carry_ref[...] = carry


def kernel(x):
    rows, n = x.shape
    grid = (n // _BLK,)
    tri = jnp.triu(jnp.ones((_SUB, _SUB), dtype=jnp.bfloat16))
    return pl.pallas_call(
        _body,
        grid=grid,
        in_specs=[
            pl.BlockSpec((rows, _BLK), lambda i: (0, i)),
            pl.BlockSpec((_SUB, _SUB), lambda i: (0, 0)),
        ],
        out_specs=pl.BlockSpec((rows, _BLK), lambda i: (0, i)),
        out_shape=jax.ShapeDtypeStruct((rows, n), jnp.float32),
        scratch_shapes=[pltpu.VMEM((rows, 1), jnp.float32)],
    )(x, tri)


# speedup vs baseline: 1.3719x; 1.3719x over previous
import jax
import jax.numpy as jnp
from jax.experimental import pallas as pl
from jax.experimental.pallas import tpu as pltpu

_ROWS = 128
_N = 32768
_BLK = 16384
_SUB = 256


def _body(x_ref, t_ref, o_ref, carry_ref):
    i = pl.program_id(0)

    @pl.when(i == 0)
    def _init():
        carry_ref[...] = jnp.zeros_like(carry_ref)

    carry = carry_ref[...]
    for k in range(_BLK // _SUB):
        x = x_ref[:, k * _SUB:(k + 1) * _SUB]
        pre = jax.lax.dot(
            x.astype(jnp.bfloat16), t_ref[...],
            preferred_element_type=jnp.float32)
        o_ref[:, k * _SUB:(k + 1) * _SUB] = pre + carry
        carry = carry + jnp.sum(x, axis=1, keepdims=True)
    carry_ref[...] = carry


def kernel(x):
    rows, n = x.shape
    grid = (n // _BLK,)
    tri = jnp.triu(jnp.ones((_SUB, _SUB), dtype=jnp.bfloat16))
    return pl.pallas_call(
        _body,
        grid=grid,
        in_specs=[
            pl.BlockSpec((rows, _BLK), lambda i: (0, i)),
            pl.BlockSpec((_SUB, _SUB), lambda i: (0, 0)),
        ],
        out_specs=pl.BlockSpec((rows, _BLK), lambda i: (0, i)),
        out_shape=jax.ShapeDtypeStruct((rows, n), jnp.float32),
        scratch_shapes=[pltpu.VMEM((rows, 1), jnp.float32)],
    )(x, tri)
